# Initial kernel scaffold; baseline (speedup 1.0000x reference)
import jax
import jax.numpy as jnp
from jax.experimental import pallas as pl


def kernel(x, weight, gamma, beta):
    raise NotImplementedError("write your pallas kernel here")


# speedup vs baseline: 1.8458x; 1.8458x over previous
import functools

import jax
import jax.numpy as jnp
from jax import lax
from jax.experimental import pallas as pl
from jax.experimental.pallas import tpu as pltpu

_EPS = 1e-5
_K = 3


def _fused_kernel(x_ref, w_ref, g_ref, b_ref, o_ref, xpad, col, acc,
                  *, cin, cout, d, h, w, td):
    hw = h * w
    l = d * hw
    lt = td * hw
    n_t = d // td
    lw = lt + 2 * hw

    zplane = jnp.zeros((cin, hw), jnp.bfloat16)
    xpad[:, 0:hw] = zplane
    xpad[:, hw + l:] = zplane
    xpad[:, hw:hw + l] = x_ref[...].astype(jnp.bfloat16)

    lane = lax.broadcasted_iota(jnp.int32, (1, lt), 1)
    w_id = lane % w
    h_id = (lane // w) % h
    h_ok = {0: h_id >= 1, 1: None, 2: h_id <= h - 2}
    w_ok = {0: w_id >= 1, 1: None, 2: w_id <= w - 2}

    s_sum = jnp.zeros((cout, 1), jnp.float32)
    s_ssq = jnp.zeros((cout, 1), jnp.float32)
    for t in range(n_t):
        xw = xpad[:, t * lt: t * lt + lw]
        for kh in range(_K):
            for kw in range(_K):
                s = (kh - 1) * w + (kw - 1)
                rolled = xw if s == 0 else pltpu.roll(xw, (-s) % lw, axis=1)
                m = h_ok[kh]
                if w_ok[kw] is not None:
                    m = w_ok[kw] if m is None else jnp.logical_and(m, w_ok[kw])
                for kd in range(_K):
                    tap = (kd * _K + kh) * _K + kw
                    row = rolled[:, kd * hw: kd * hw + lt]
                    if m is not None:
                        row = jnp.where(m, row, jnp.bfloat16(0))
                    col[tap * cin:(tap + 1) * cin, :] = row
        conv_t = jnp.dot(w_ref[...], col[...],
                         preferred_element_type=jnp.float32)
        acc[:, t * lt:(t + 1) * lt] = conv_t
        s_sum = s_sum + jnp.sum(conv_t, axis=1, keepdims=True)
        s_ssq = s_ssq + jnp.sum(conv_t * conv_t, axis=1, keepdims=True)

    inv = 1.0 / float(l)
    mean = s_sum * inv
    var = s_ssq * inv - mean * mean
    scale = g_ref[...] * lax.rsqrt(var + _EPS)
    bias = b_ref[...] - mean * scale
    o_ref[...] = jnp.maximum(acc[...] * scale + bias, 0.0).astype(o_ref.dtype)


@jax.jit
def _conv_block(x, weight, gamma, beta):
    n, cin, d, h, w = x.shape
    cout = weight.shape[0]
    hw = h * w
    td = 8 if d % 8 == 0 else d
    taps = _K * _K * _K

    x_flat = x.reshape(n, cin, d * hw)
    w2 = jnp.transpose(weight, (0, 2, 3, 4, 1)).reshape(cout, taps * cin)
    w2 = w2.astype(jnp.bfloat16)
    g2 = gamma.reshape(cout, 1).astype(jnp.float32)
    b2 = beta.reshape(cout, 1).astype(jnp.float32)

    body = functools.partial(_fused_kernel, cin=cin, cout=cout,
                             d=d, h=h, w=w, td=td)
    out = pl.pallas_call(
        body,
        out_shape=jax.ShapeDtypeStruct((n, cout, d * hw), x.dtype),
        grid=(n,),
        in_specs=[
            pl.BlockSpec((None, cin, d * hw), lambda b: (b, 0, 0)),
            pl.BlockSpec((cout, taps * cin), lambda b: (0, 0)),
            pl.BlockSpec((cout, 1), lambda b: (0, 0)),
            pl.BlockSpec((cout, 1), lambda b: (0, 0)),
        ],
        out_specs=pl.BlockSpec((None, cout, d * hw), lambda b: (b, 0, 0)),
        scratch_shapes=[
            pltpu.VMEM((cin, (d + 2) * hw), jnp.bfloat16),
            pltpu.VMEM((taps * cin, td * hw), jnp.bfloat16),
            pltpu.VMEM((cout, d * hw), jnp.float32),
        ],
        compiler_params=pltpu.CompilerParams(
            dimension_semantics=("parallel",),
            vmem_limit_bytes=48 * 1024 * 1024,
        ),
    )(x_flat, w2, g2, b2)
    return out.reshape(n, cout, d, h, w)


def kernel(x, weight, gamma, beta):
    return _conv_block(x, weight, gamma, beta)


# speedup vs baseline: 2.6502x; 1.4358x over previous
import functools

import jax
import jax.numpy as jnp
from jax import lax
from jax.experimental import pallas as pl
from jax.experimental.pallas import tpu as pltpu

_EPS = 1e-5
_K = 3


def _fused_kernel(x_ref, w_ref, g_ref, b_ref, o_ref, xpad, col, acc,
                  *, cin, cout, d, h, w, td):
    hw = h * w
    l = d * hw
    lt = td * hw
    n_t = d // td
    lw = lt + 2 * hw

    zplane = jnp.zeros((cin, hw), jnp.bfloat16)
    xpad[:, 0:hw] = zplane
    xpad[:, hw + l:] = zplane
    xflat = x_ref[...].astype(jnp.bfloat16).reshape(cin, l)
    xpad[:, hw:hw + l] = xflat

    lane = lax.broadcasted_iota(jnp.int32, (1, lt), 1)
    w_id = lane % w
    h_id = (lane // w) % h
    h_ok = {0: h_id >= 1, 1: None, 2: h_id <= h - 2}
    w_ok = {0: w_id >= 1, 1: None, 2: w_id <= w - 2}

    s_sum = jnp.zeros((cout, 1), jnp.float32)
    s_ssq = jnp.zeros((cout, 1), jnp.float32)
    for t in range(n_t):
        xw = xpad[:, t * lt: t * lt + lw]
        for kh in range(_K):
            for kw in range(_K):
                s = (kh - 1) * w + (kw - 1)
                rolled = xw if s == 0 else pltpu.roll(xw, (-s) % lw, axis=1)
                m = h_ok[kh]
                if w_ok[kw] is not None:
                    m = w_ok[kw] if m is None else jnp.logical_and(m, w_ok[kw])
                for kd in range(_K):
                    tap = (kd * _K + kh) * _K + kw
                    row = rolled[:, kd * hw: kd * hw + lt]
                    if m is not None:
                        row = jnp.where(m, row, jnp.bfloat16(0))
                    col[tap * cin:(tap + 1) * cin, :] = row
        conv_t = jnp.dot(w_ref[...], col[...],
                         preferred_element_type=jnp.float32)
        acc[:, t * lt:(t + 1) * lt] = conv_t
        s_sum = s_sum + jnp.sum(conv_t, axis=1, keepdims=True)
        s_ssq = s_ssq + jnp.sum(conv_t * conv_t, axis=1, keepdims=True)

    inv = 1.0 / float(l)
    mean = s_sum * inv
    var = s_ssq * inv - mean * mean
    scale = g_ref[...] * lax.rsqrt(var + _EPS)
    bias = b_ref[...] - mean * scale
    y = jnp.maximum(acc[...] * scale + bias, 0.0).astype(o_ref.dtype)
    o_ref[...] = y.reshape(cout, d * h, w)


@jax.jit
def _conv_block(x, weight, gamma, beta):
    n, cin, d, h, w = x.shape
    cout = weight.shape[0]
    hw = h * w
    td = 8 if d % 8 == 0 else d
    taps = _K * _K * _K

    x4 = x.reshape(n, cin, d * h, w)
    w2 = jnp.transpose(weight, (0, 2, 3, 4, 1)).reshape(cout, taps * cin)
    w2 = w2.astype(jnp.bfloat16)
    g2 = gamma.reshape(cout, 1).astype(jnp.float32)
    b2 = beta.reshape(cout, 1).astype(jnp.float32)

    body = functools.partial(_fused_kernel, cin=cin, cout=cout,
                             d=d, h=h, w=w, td=td)
    out = pl.pallas_call(
        body,
        out_shape=jax.ShapeDtypeStruct((n, cout, d * h, w), x.dtype),
        grid=(n,),
        in_specs=[
            pl.BlockSpec((None, cin, d * h, w), lambda b: (b, 0, 0, 0)),
            pl.BlockSpec((cout, taps * cin), lambda b: (0, 0)),
            pl.BlockSpec((cout, 1), lambda b: (0, 0)),
            pl.BlockSpec((cout, 1), lambda b: (0, 0)),
        ],
        out_specs=pl.BlockSpec((None, cout, d * h, w), lambda b: (b, 0, 0, 0)),
        scratch_shapes=[
            pltpu.VMEM((cin, (d + 2) * hw), jnp.bfloat16),
            pltpu.VMEM((taps * cin, td * hw), jnp.bfloat16),
            pltpu.VMEM((cout, d * hw), jnp.float32),
        ],
        compiler_params=pltpu.CompilerParams(
            dimension_semantics=("parallel",),
            vmem_limit_bytes=60 * 1024 * 1024,
        ),
    )(x4, w2, g2, b2)
    return out.reshape(n, cout, d, h, w)


def kernel(x, weight, gamma, beta):
    return _conv_block(x, weight, gamma, beta)


# speedup vs baseline: 2.6590x; 1.0033x over previous
import functools

import jax
import jax.numpy as jnp
from jax import lax
from jax.experimental import pallas as pl
from jax.experimental.pallas import tpu as pltpu

_EPS = 1e-5
_K = 3


def _fused_kernel(x_ref, w_ref, g_ref, b_ref, o_ref, xpad, col, acc,
                  *, cin, cout, d, h, w, td):
    hw = h * w
    l = d * hw
    lt = td * hw
    n_t = d // td
    lw = lt + 2 * hw

    zplane = jnp.zeros((cin, hw), jnp.bfloat16)
    xpad[:, 0:hw] = zplane
    xpad[:, hw + l:] = zplane
    xflat = x_ref[...].astype(jnp.bfloat16).reshape(cin, l)
    xpad[:, hw:hw + l] = xflat

    lane = lax.broadcasted_iota(jnp.int32, (1, lw), 1)
    w_id = lane % w
    h_id = (lane // w) % h
    h_ok = {0: h_id >= 1, 1: None, 2: h_id <= h - 2}
    w_ok = {0: w_id >= 1, 1: None, 2: w_id <= w - 2}

    s_sum = jnp.zeros((cout, 1), jnp.float32)
    s_ssq = jnp.zeros((cout, 1), jnp.float32)
    for t in range(n_t):
        xw = xpad[:, t * lt: t * lt + lw]
        for kh in range(_K):
            for kw in range(_K):
                s = (kh - 1) * w + (kw - 1)
                rolled = xw if s == 0 else pltpu.roll(xw, (-s) % lw, axis=1)
                m = h_ok[kh]
                if w_ok[kw] is not None:
                    m = w_ok[kw] if m is None else jnp.logical_and(m, w_ok[kw])
                if m is not None:
                    rolled = jnp.where(m, rolled, jnp.bfloat16(0))
                for kd in range(_K):
                    tap = (kd * _K + kh) * _K + kw
                    col[tap * cin:(tap + 1) * cin, :] = \
                        rolled[:, kd * hw: kd * hw + lt]
        conv_t = jnp.dot(w_ref[...], col[...],
                         preferred_element_type=jnp.float32)
        acc[:, t * lt:(t + 1) * lt] = conv_t
        s_sum = s_sum + jnp.sum(conv_t, axis=1, keepdims=True)
        s_ssq = s_ssq + jnp.sum(conv_t * conv_t, axis=1, keepdims=True)

    inv = 1.0 / float(l)
    mean = s_sum * inv
    var = s_ssq * inv - mean * mean
    scale = g_ref[...] * lax.rsqrt(var + _EPS)
    bias = b_ref[...] - mean * scale
    y = jnp.maximum(acc[...] * scale + bias, 0.0).astype(o_ref.dtype)
    o_ref[...] = y.reshape(cout, d * h, w)


@jax.jit
def _conv_block(x, weight, gamma, beta):
    n, cin, d, h, w = x.shape
    cout = weight.shape[0]
    hw = h * w
    td = 8 if d % 8 == 0 else d
    taps = _K * _K * _K

    x4 = x.reshape(n, cin, d * h, w)
    w2 = jnp.transpose(weight, (0, 2, 3, 4, 1)).reshape(cout, taps * cin)
    w2 = w2.astype(jnp.bfloat16)
    g2 = gamma.reshape(cout, 1).astype(jnp.float32)
    b2 = beta.reshape(cout, 1).astype(jnp.float32)

    body = functools.partial(_fused_kernel, cin=cin, cout=cout,
                             d=d, h=h, w=w, td=td)
    out = pl.pallas_call(
        body,
        out_shape=jax.ShapeDtypeStruct((n, cout, d * h, w), x.dtype),
        grid=(n,),
        in_specs=[
            pl.BlockSpec((None, cin, d * h, w), lambda b: (b, 0, 0, 0)),
            pl.BlockSpec((cout, taps * cin), lambda b: (0, 0)),
            pl.BlockSpec((cout, 1), lambda b: (0, 0)),
            pl.BlockSpec((cout, 1), lambda b: (0, 0)),
        ],
        out_specs=pl.BlockSpec((None, cout, d * h, w), lambda b: (b, 0, 0, 0)),
        scratch_shapes=[
            pltpu.VMEM((cin, (d + 2) * hw), jnp.bfloat16),
            pltpu.VMEM((taps * cin, td * hw), jnp.bfloat16),
            pltpu.VMEM((cout, d * hw), jnp.float32),
        ],
        compiler_params=pltpu.CompilerParams(
            dimension_semantics=("parallel",),
            vmem_limit_bytes=60 * 1024 * 1024,
        ),
    )(x4, w2, g2, b2)
    return out.reshape(n, cout, d, h, w)


def kernel(x, weight, gamma, beta):
    return _conv_block(x, weight, gamma, beta)


# speedup vs baseline: 2.7814x; 1.0460x over previous
import functools

import jax
import jax.numpy as jnp
from jax import lax
from jax.experimental import pallas as pl
from jax.experimental.pallas import tpu as pltpu

_EPS = 1e-5
_K = 3


def _fused_kernel(x_ref, w_ref, g_ref, b_ref, o_ref, xpad, col, acc,
                  *, cin, cout, d, h, w, td):
    hw = h * w
    l = d * hw
    lt = td * hw
    n_t = d // td
    lw = lt + 2 * hw

    zplane = jnp.zeros((cin, hw), jnp.bfloat16)
    xpad[:, 0:hw] = zplane
    xpad[:, hw + l:] = zplane
    xflat = x_ref[...].astype(jnp.bfloat16).reshape(cin, l)
    xpad[:, hw:hw + l] = xflat

    lane = lax.broadcasted_iota(jnp.int32, (1, lw), 1)
    w_id = lane % w
    h_id = (lane // w) % h
    h_ok = {0: h_id >= 1, 1: None, 2: h_id <= h - 2}
    w_ok = {0: w_id >= 1, 1: None, 2: w_id <= w - 2}

    s_sum = jnp.zeros((cout, 1), jnp.float32)
    s_ssq = jnp.zeros((cout, 1), jnp.float32)
    for t in range(n_t):
        xw = xpad[:, t * lt: t * lt + lw]
        for kh in range(_K):
            for kw in range(_K):
                s = (kh - 1) * w + (kw - 1)
                rolled = xw if s == 0 else pltpu.roll(xw, (-s) % lw, axis=1)
                m = h_ok[kh]
                if w_ok[kw] is not None:
                    m = w_ok[kw] if m is None else jnp.logical_and(m, w_ok[kw])
                if m is not None:
                    rolled = jnp.where(m, rolled, jnp.bfloat16(0))
                for kd in range(_K):
                    tap = (kd * _K + kh) * _K + kw
                    col[tap * cin:(tap + 1) * cin, :] = \
                        rolled[:, kd * hw: kd * hw + lt]
        conv_t = jnp.dot(w_ref[...], col[...],
                         preferred_element_type=jnp.float32)
        acc[:, t * lt:(t + 1) * lt] = conv_t
        s_sum = s_sum + jnp.sum(conv_t, axis=1, keepdims=True)
        s_ssq = s_ssq + jnp.sum(conv_t * conv_t, axis=1, keepdims=True)

    inv = 1.0 / float(l)
    mean = s_sum * inv
    var = s_ssq * inv - mean * mean
    scale = g_ref[...] * lax.rsqrt(var + _EPS)
    bias = b_ref[...] - mean * scale
    y = jnp.maximum(acc[...] * scale + bias, 0.0).astype(o_ref.dtype)
    o_ref[...] = y.reshape(cout, d * h, w)


@jax.jit
def _conv_block(x, weight, gamma, beta):
    n, cin, d, h, w = x.shape
    cout = weight.shape[0]
    hw = h * w
    td = 16 if d % 16 == 0 else d
    taps = _K * _K * _K

    x4 = x.reshape(n, cin, d * h, w)
    w2 = jnp.transpose(weight, (0, 2, 3, 4, 1)).reshape(cout, taps * cin)
    w2 = w2.astype(jnp.bfloat16)
    g2 = gamma.reshape(cout, 1).astype(jnp.float32)
    b2 = beta.reshape(cout, 1).astype(jnp.float32)

    body = functools.partial(_fused_kernel, cin=cin, cout=cout,
                             d=d, h=h, w=w, td=td)
    out = pl.pallas_call(
        body,
        out_shape=jax.ShapeDtypeStruct((n, cout, d * h, w), x.dtype),
        grid=(n,),
        in_specs=[
            pl.BlockSpec((None, cin, d * h, w), lambda b: (b, 0, 0, 0)),
            pl.BlockSpec((cout, taps * cin), lambda b: (0, 0)),
            pl.BlockSpec((cout, 1), lambda b: (0, 0)),
            pl.BlockSpec((cout, 1), lambda b: (0, 0)),
        ],
        out_specs=pl.BlockSpec((None, cout, d * h, w), lambda b: (b, 0, 0, 0)),
        scratch_shapes=[
            pltpu.VMEM((cin, (d + 2) * hw), jnp.bfloat16),
            pltpu.VMEM((taps * cin, td * hw), jnp.bfloat16),
            pltpu.VMEM((cout, d * hw), jnp.float32),
        ],
        compiler_params=pltpu.CompilerParams(
            dimension_semantics=("parallel",),
            vmem_limit_bytes=60 * 1024 * 1024,
        ),
    )(x4, w2, g2, b2)
    return out.reshape(n, cout, d, h, w)


def kernel(x, weight, gamma, beta):
    return _conv_block(x, weight, gamma, beta)
